# Initial kernel scaffold; baseline (speedup 1.0000x reference)
import jax
import jax.numpy as jnp
from jax.experimental import pallas as pl


def kernel(x, edge_index, W1, b1, W2, b2, Wm1, bm1, Wm2, bm2):
    raise NotImplementedError("write your pallas kernel here")


# speedup vs baseline: 9.4056x; 9.4056x over previous
import functools

import jax
import jax.numpy as jnp
from jax import lax
from jax.experimental import pallas as pl
from jax.experimental.pallas import tpu as pltpu
from jax.experimental.pallas import tpu_sc as plsc

NC, NS, LANES = 2, 16, 16
NW = NC * NS
N_PAD = 10240
NPT = N_PAD // NW
NPS = N_PAD // NS
H = 32
CHUNK = 128
BM = 1024


def _mesh():
    return plsc.VectorSubcoreMesh(core_axis_name="c", subcore_axis_name="s")


def _tc_mlp(x_pad, W1, b1, W2, b2, d0_2d, d1_2d):
    F = x_pad.shape[1]
    Hh = W1.shape[1]
    BR = BM // 128

    def body(x_ref, w1_ref, b1_ref, w2_ref, b2_ref, d0_ref, d1_ref,
             h_ref, dinv_ref):
        h1 = jnp.dot(x_ref[...], w1_ref[...], preferred_element_type=jnp.float32)
        h1 = jnp.maximum(h1 + b1_ref[...], 0.0)
        h2 = jnp.dot(h1, w2_ref[...], preferred_element_type=jnp.float32)
        h_ref[...] = jnp.maximum(h2 + b2_ref[...], 0.0)
        deg = jnp.maximum(d0_ref[...] + d1_ref[...], 1.0)
        dinv_ref[...] = lax.rsqrt(deg)

    return pl.pallas_call(
        body,
        grid=(N_PAD // BM,),
        in_specs=[
            pl.BlockSpec((BM, F), lambda i: (i, 0)),
            pl.BlockSpec((F, Hh), lambda i: (0, 0)),
            pl.BlockSpec((1, Hh), lambda i: (0, 0)),
            pl.BlockSpec((Hh, Hh), lambda i: (0, 0)),
            pl.BlockSpec((1, Hh), lambda i: (0, 0)),
            pl.BlockSpec((BR, 128), lambda i: (i, 0)),
            pl.BlockSpec((BR, 128), lambda i: (i, 0)),
        ],
        out_specs=[
            pl.BlockSpec((BM, Hh), lambda i: (i, 0)),
            pl.BlockSpec((BR, 128), lambda i: (i, 0)),
        ],
        out_shape=[
            jax.ShapeDtypeStruct((N_PAD, Hh), jnp.float32),
            jax.ShapeDtypeStruct((N_PAD // 128, 128), jnp.float32),
        ],
    )(x_pad, W1, b1, W2, b2, d0_2d, d1_2d)


def _tc_head(h, p1, p2, A0, A1, A2, bm1, Wm2, bm2):
    C = Wm2.shape[1]

    def body(h_ref, p1_ref, p2_ref, a0, a1, a2, b1r, w2r, b2r, o_ref):
        t = jnp.dot(h_ref[...], a0[...], preferred_element_type=jnp.float32)
        t += jnp.dot(p1_ref[...], a1[...], preferred_element_type=jnp.float32)
        t += jnp.dot(p2_ref[...], a2[...], preferred_element_type=jnp.float32)
        z = jnp.maximum(t + b1r[...], 0.0)
        o_ref[...] = jnp.dot(z, w2r[...], preferred_element_type=jnp.float32) + b2r[...]

    return pl.pallas_call(
        body,
        grid=(N_PAD // BM,),
        in_specs=[
            pl.BlockSpec((BM, H), lambda i: (i, 0)),
            pl.BlockSpec((BM, H), lambda i: (i, 0)),
            pl.BlockSpec((BM, H), lambda i: (i, 0)),
            pl.BlockSpec((H, H), lambda i: (0, 0)),
            pl.BlockSpec((H, H), lambda i: (0, 0)),
            pl.BlockSpec((H, H), lambda i: (0, 0)),
            pl.BlockSpec((1, H), lambda i: (0, 0)),
            pl.BlockSpec((H, C), lambda i: (0, 0)),
            pl.BlockSpec((1, C), lambda i: (0, 0)),
        ],
        out_specs=pl.BlockSpec((BM, C), lambda i: (i, 0)),
        out_shape=jax.ShapeDtypeStruct((N_PAD, C), jnp.float32),
    )(h, p1, p2, A0, A1, A2, bm1, Wm2, bm2)


def _build_deg(K):
    @functools.partial(
        pl.kernel,
        mesh=_mesh(),
        compiler_params=pltpu.CompilerParams(use_tc_tiling_on_sc=False),
        out_type=(
            jax.ShapeDtypeStruct((N_PAD,), jnp.float32),
            jax.ShapeDtypeStruct((N_PAD,), jnp.float32),
        ),
        scratch_types=[
            pltpu.VMEM((K, CHUNK), jnp.int32),
            pltpu.VMEM((CHUNK,), jnp.float32),
            pltpu.VMEM((NPS,), jnp.float32),
            pltpu.VMEM_SHARED((N_PAD,), jnp.float32),
        ],
    )
    def deg_kernel(dst_hbm, d0_out, d1_out, didx, ones_v, buf, deg_sh):
        c = lax.axis_index("c")
        s = lax.axis_index("s")
        wid = c * NS + s
        pltpu.sync_copy(dst_hbm.at[wid], didx)
        zeros16 = jnp.zeros((LANES,), jnp.float32)

        def zf(i, carry):
            buf[pl.ds(i * LANES, LANES)] = zeros16
            return carry
        lax.fori_loop(0, NPS // LANES, zf, 0)

        def of(i, carry):
            ones_v[pl.ds(i * LANES, LANES)] = zeros16 + 1.0
            return carry
        lax.fori_loop(0, CHUNK // LANES, of, 0)

        pltpu.sync_copy(buf, deg_sh.at[pl.ds(s * NPS, NPS)])
        plsc.subcore_barrier()

        def body(k, carry):
            pltpu.sync_copy(ones_v, deg_sh.at[didx.at[k]], add=True)
            return carry
        lax.fori_loop(0, K, body, 0)

        plsc.subcore_barrier()
        pltpu.sync_copy(deg_sh.at[pl.ds(s * NPS, NPS)], buf)

        @pl.when(c == 0)
        def _():
            pltpu.sync_copy(buf, d0_out.at[pl.ds(s * NPS, NPS)])

        @pl.when(c == 1)
        def _():
            pltpu.sync_copy(buf, d1_out.at[pl.ds(s * NPS, NPS)])

    return deg_kernel


def _build_scale0():
    @functools.partial(
        pl.kernel,
        mesh=_mesh(),
        compiler_params=pltpu.CompilerParams(use_tc_tiling_on_sc=False),
        out_type=jax.ShapeDtypeStruct((N_PAD, H), jnp.float32),
        scratch_types=[
            pltpu.VMEM((NPT, H), jnp.float32),
            pltpu.VMEM((NPT, H), jnp.float32),
            pltpu.VMEM((NPT,), jnp.float32),
        ],
    )
    def scale0(h_hbm, dinv_hbm, q_out, h_v, q_v, dv):
        c = lax.axis_index("c")
        s = lax.axis_index("s")
        base = (c * NS + s) * NPT
        pltpu.sync_copy(h_hbm.at[pl.ds(base, NPT)], h_v)
        pltpu.sync_copy(dinv_hbm.at[pl.ds(base, NPT)], dv)

        def grp(g, carry):
            y16 = dv[pl.ds(g * LANES, LANES)]
            for i in range(LANES):
                r = g * LANES + i
                yb = jnp.full((LANES,), y16[i], jnp.float32)
                q_v[r, pl.ds(0, LANES)] = h_v[r, pl.ds(0, LANES)] * yb
                q_v[r, pl.ds(LANES, LANES)] = h_v[r, pl.ds(LANES, LANES)] * yb
            return carry
        lax.fori_loop(0, NPT // LANES, grp, 0)

        pltpu.sync_copy(q_v, q_out.at[pl.ds(base, NPT)])

    return scale0


def _build_edges(K):
    @functools.partial(
        pl.kernel,
        mesh=_mesh(),
        compiler_params=pltpu.CompilerParams(use_tc_tiling_on_sc=False),
        out_type=(
            jax.ShapeDtypeStruct((N_PAD, H), jnp.float32),
            jax.ShapeDtypeStruct((N_PAD, H), jnp.float32),
        ),
        scratch_types=[
            pltpu.VMEM((K, CHUNK), jnp.int32),
            pltpu.VMEM((K, CHUNK), jnp.int32),
            pltpu.VMEM((CHUNK, H), jnp.float32),
            pltpu.VMEM_SHARED((N_PAD, H), jnp.float32),
            pltpu.SemaphoreType.DMA,
        ],
    )
    def edges(q_hbm, src_hbm, dst_hbm, a0_out, a1_out, sidx, didx, rows, agg_sh, sem):
        c = lax.axis_index("c")
        s = lax.axis_index("s")
        wid = c * NS + s
        pltpu.sync_copy(src_hbm.at[wid], sidx)
        pltpu.sync_copy(dst_hbm.at[wid], didx)

        zeros16 = jnp.zeros((LANES,), jnp.float32)

        def zf(i, carry):
            rows[i, pl.ds(0, LANES)] = zeros16
            rows[i, pl.ds(LANES, LANES)] = zeros16
            return carry
        lax.fori_loop(0, CHUNK, zf, 0)

        def zs(j, carry):
            pltpu.sync_copy(rows, agg_sh.at[pl.ds(s * NPS + j * CHUNK, CHUNK)])
            return carry
        lax.fori_loop(0, NPS // CHUNK, zs, 0)

        plsc.subcore_barrier()

        def body(k, carry):
            pltpu.async_copy(q_hbm.at[sidx.at[k]], rows, sem).wait()
            pltpu.sync_copy(rows, agg_sh.at[didx.at[k]], add=True)
            return carry
        lax.fori_loop(0, K, body, 0)

        plsc.subcore_barrier()

        @pl.when(c == 0)
        def _():
            def dump(j, carry):
                pltpu.sync_copy(agg_sh.at[pl.ds(s * NPS + j * CHUNK, CHUNK)], rows)
                pltpu.sync_copy(rows, a0_out.at[pl.ds(s * NPS + j * CHUNK, CHUNK)])
                return carry
            lax.fori_loop(0, NPS // CHUNK, dump, 0)

        @pl.when(c == 1)
        def _():
            def dump(j, carry):
                pltpu.sync_copy(agg_sh.at[pl.ds(s * NPS + j * CHUNK, CHUNK)], rows)
                pltpu.sync_copy(rows, a1_out.at[pl.ds(s * NPS + j * CHUNK, CHUNK)])
                return carry
            lax.fori_loop(0, NPS // CHUNK, dump, 0)

    return edges


def _build_combine():
    @functools.partial(
        pl.kernel,
        mesh=_mesh(),
        compiler_params=pltpu.CompilerParams(use_tc_tiling_on_sc=False),
        out_type=(
            jax.ShapeDtypeStruct((N_PAD, H), jnp.float32),
            jax.ShapeDtypeStruct((N_PAD, H), jnp.float32),
        ),
        scratch_types=[
            pltpu.VMEM((NPT, H), jnp.float32),
            pltpu.VMEM((NPT, H), jnp.float32),
            pltpu.VMEM((NPT, H), jnp.float32),
            pltpu.VMEM((NPT, H), jnp.float32),
            pltpu.VMEM((NPT,), jnp.float32),
        ],
    )
    def combine(p_hbm, a0_hbm, a1_hbm, dinv_hbm, p_out, q_out, p_v, a_v, b_v, q_v, dv):
        c = lax.axis_index("c")
        s = lax.axis_index("s")
        base = (c * NS + s) * NPT
        pltpu.sync_copy(p_hbm.at[pl.ds(base, NPT)], p_v)
        pltpu.sync_copy(a0_hbm.at[pl.ds(base, NPT)], a_v)
        pltpu.sync_copy(a1_hbm.at[pl.ds(base, NPT)], b_v)
        pltpu.sync_copy(dinv_hbm.at[pl.ds(base, NPT)], dv)

        def grp(g, carry):
            y16 = dv[pl.ds(g * LANES, LANES)]
            for i in range(LANES):
                r = g * LANES + i
                yb = jnp.full((LANES,), y16[i], jnp.float32)
                for half in (0, LANES):
                    agg = a_v[r, pl.ds(half, LANES)] + b_v[r, pl.ds(half, LANES)]
                    p = p_v[r, pl.ds(half, LANES)] - yb * agg
                    p_v[r, pl.ds(half, LANES)] = p
                    q_v[r, pl.ds(half, LANES)] = p * yb
            return carry
        lax.fori_loop(0, NPT // LANES, grp, 0)

        pltpu.sync_copy(p_v, p_out.at[pl.ds(base, NPT)])
        pltpu.sync_copy(q_v, q_out.at[pl.ds(base, NPT)])

    return combine


def kernel(x, edge_index, W1, b1, W2, b2, Wm1, bm1, Wm2, bm2):
    N = x.shape[0]
    Hh = W1.shape[1]
    E = edge_index.shape[1]
    ept = -(-E // NW)
    K = -(-ept // CHUNK)
    E_PAD = K * CHUNK * NW

    src = edge_index[0]
    dst = edge_index[1]
    src_p = jnp.pad(src, (0, E_PAD - E)).reshape(NW, K, CHUNK)
    dst_p = jnp.pad(dst, (0, E_PAD - E), constant_values=N_PAD - 1).reshape(NW, K, CHUNK)
    x_pad = jnp.pad(x, ((0, N_PAD - N), (0, 0)))

    d0, d1 = _build_deg(K)(dst_p)
    h, dinv2d = _tc_mlp(x_pad, W1, b1.reshape(1, -1), W2, b2.reshape(1, -1),
                        d0.reshape(-1, 128), d1.reshape(-1, 128))
    dinv = dinv2d.reshape(N_PAD)
    q0 = _build_scale0()(h, dinv)
    a0, a1 = _build_edges(K)(q0, src_p, dst_p)
    p1, q1 = _build_combine()(h, a0, a1, dinv)
    b0, b1_agg = _build_edges(K)(q1, src_p, dst_p)
    p2, _ = _build_combine()(p1, b0, b1_agg, dinv)

    Wa, Wb, Wc = Wm1[0:Hh], Wm1[Hh:2 * Hh], Wm1[2 * Hh:3 * Hh]
    A0 = 3.0 * Wa
    A1 = -3.0 * Wa + 3.0 * Wb
    A2 = 0.75 * Wa - 1.5 * Wb + 0.75 * Wc

    out = _tc_head(h, p1, p2, A0, A1, A2,
                   bm1.reshape(1, -1), Wm2, bm2.reshape(1, -1))
    return out[:N]
